# baseline (speedup 1.0000x reference)
import jax
import jax.numpy as jnp
from jax.experimental import pallas as pl
from jax.experimental.pallas import tpu as pltpu

N = 4096
F = 128
HH = 128
WW = 128
NPIX = HH * WW


def _input_kernel(x_ref, w_ref, b_ref, s0_ref, h_ref):
    acc = jnp.dot(x_ref[...], w_ref[...], preferred_element_type=jnp.float32)
    acc = jnp.maximum(acc + b_ref[...], 0.0)
    s0_ref[...] = acc[:, :64]
    h_ref[...] = acc[:, 64:].astype(jnp.bfloat16)


def _input_transform(Q, Wcat, bcat):
    BM = 512
    return pl.pallas_call(
        _input_kernel,
        grid=(N // BM,),
        in_specs=[
            pl.BlockSpec((BM, F), lambda i: (i, 0)),
            pl.BlockSpec((F, 192), lambda i: (0, 0)),
            pl.BlockSpec((1, 192), lambda i: (0, 0)),
        ],
        out_specs=[
            pl.BlockSpec((BM, 64), lambda i: (i, 0)),
            pl.BlockSpec((BM, 128), lambda i: (i, 0)),
        ],
        out_shape=[
            jax.ShapeDtypeStruct((N, 64), jnp.float32),
            jax.ShapeDtypeStruct((N, 128), jnp.bfloat16),
        ],
    )(Q, Wcat, bcat)


_BM = 256
_NB = N // _BM


def _stage_kernel(a_ref, x_hbm, y_ref, z_ref, abf_ref, xbf_ref, tbf_ref, sem):
    i = pl.program_id(0)

    @pl.when(i == 0)
    def _():
        cp = pltpu.make_async_copy(x_hbm, xbf_ref, sem)
        cp.start()
        cp.wait()

    @pl.when(i < _NB)
    def _():
        ab = a_ref[...].astype(jnp.bfloat16)
        abf_ref[pl.ds(i * _BM, _BM), :] = ab
        yb = jnp.dot(ab, xbf_ref[...], preferred_element_type=jnp.float32)
        y_ref[...] = yb
        tbf_ref[pl.ds(i * _BM, _BM), :] = yb[:, 64:].astype(jnp.bfloat16)

    @pl.when(i >= _NB)
    def _():
        j = i - _NB
        z_ref[...] = jnp.dot(abf_ref[pl.ds(j * _BM, _BM), :], tbf_ref[...],
                             preferred_element_type=jnp.float32)


def _stage(A, Xbf):
    return pl.pallas_call(
        _stage_kernel,
        grid=(2 * _NB,),
        in_specs=[
            pl.BlockSpec((_BM, N), lambda i: (jnp.minimum(i, _NB - 1), 0)),
            pl.BlockSpec(memory_space=pl.ANY),
        ],
        out_specs=[
            pl.BlockSpec((_BM, 128), lambda i: (jnp.minimum(i, _NB - 1), 0)),
            pl.BlockSpec((_BM, 64), lambda i: (jnp.maximum(i - _NB, 0), 0)),
        ],
        out_shape=[
            jax.ShapeDtypeStruct((N, 128), jnp.float32),
            jax.ShapeDtypeStruct((N, 64), jnp.float32),
        ],
        scratch_shapes=[
            pltpu.VMEM((N, N), jnp.bfloat16),
            pltpu.VMEM((N, 128), jnp.bfloat16),
            pltpu.VMEM((N, 64), jnp.bfloat16),
            pltpu.SemaphoreType.DMA,
        ],
    )(A, Xbf)


def _amp_kernel(f1_ref, f2_ref, o_ref):
    f1 = f1_ref[...]
    f2 = f2_ref[...]
    n1 = jnp.maximum(jnp.sqrt(jnp.sum(f1 * f1, axis=0)), 1e-8)
    n2 = jnp.maximum(jnp.sqrt(jnp.sum(f2 * f2, axis=0)), 1e-8)
    cs = jnp.sum(f1 * f2, axis=0) / (n1 * n2)
    o_ref[...] = jax.nn.sigmoid(1.0 - cs)[None, :]


def _amp(f11, f21):
    return pl.pallas_call(
        _amp_kernel,
        out_shape=jax.ShapeDtypeStruct((1, 192), jnp.float32),
    )(f11, f21)


def _fg_kernel(f_ref, amp_ref, w_ref, g64_ref, gh_ref):
    acc = jnp.dot(f_ref[...] * amp_ref[...], w_ref[...],
                  preferred_element_type=jnp.float32)
    g64_ref[...] = acc[:, :64]
    gh_ref[...] = acc[:, 64:].astype(jnp.bfloat16)


def _fg(f, amp, W2cat):
    BM = 512
    return pl.pallas_call(
        _fg_kernel,
        grid=(N // BM,),
        in_specs=[
            pl.BlockSpec((BM, 192), lambda i: (i, 0)),
            pl.BlockSpec((1, 192), lambda i: (0, 0)),
            pl.BlockSpec((192, 192), lambda i: (0, 0)),
        ],
        out_specs=[
            pl.BlockSpec((BM, 64), lambda i: (i, 0)),
            pl.BlockSpec((BM, 128), lambda i: (i, 0)),
        ],
        out_shape=[
            jax.ShapeDtypeStruct((N, 64), jnp.float32),
            jax.ShapeDtypeStruct((N, 128), jnp.bfloat16),
        ],
    )(f, amp, W2cat)


def _mm_kernel(x_ref, w_ref, o_ref):
    o_ref[...] = jnp.dot(x_ref[...], w_ref[...], preferred_element_type=jnp.float32)


def _feats_project(feats0, Wp):
    BM = 512
    return pl.pallas_call(
        _mm_kernel,
        grid=(N // BM,),
        in_specs=[
            pl.BlockSpec((BM, 384), lambda i: (i, 0)),
            pl.BlockSpec((384, 32), lambda i: (0, 0)),
        ],
        out_specs=pl.BlockSpec((BM, 32), lambda i: (i, 0)),
        out_shape=jax.ShapeDtypeStruct((N, 32), jnp.float32),
    )(feats0, Wp)


def _s_kernel(s_ref, f_hbm, c_hbm, o_ref, fs_ref, cs_ref, sem1, sem2):
    i = pl.program_id(0)

    @pl.when(i == 0)
    def _():
        cp1 = pltpu.make_async_copy(f_hbm, fs_ref, sem1)
        cp1.start()
        cp2 = pltpu.make_async_copy(c_hbm, cs_ref, sem2)
        cp2.start()
        cp1.wait()
        cp2.wait()

    y = jnp.dot(s_ref[...], fs_ref[...], preferred_element_type=jnp.float32)
    y = y + cs_ref[...]
    o_ref[...] = jnp.where(y >= 0, y, 0.01 * y)


def _s_matmul(S, F2, cp):
    BM = 512
    return pl.pallas_call(
        _s_kernel,
        grid=(NPIX // BM,),
        in_specs=[
            pl.BlockSpec((BM, N), lambda i: (i, 0)),
            pl.BlockSpec(memory_space=pl.ANY),
            pl.BlockSpec(memory_space=pl.ANY),
        ],
        out_specs=pl.BlockSpec((BM, 32), lambda i: (i, 0)),
        out_shape=jax.ShapeDtypeStruct((NPIX, 32), jnp.float32),
        scratch_shapes=[
            pltpu.VMEM((N, 32), jnp.float32),
            pltpu.VMEM((1, 32), jnp.float32),
            pltpu.SemaphoreType.DMA,
            pltpu.SemaphoreType.DMA,
        ],
    )(S, F2, cp)


def _head_kernel(x_ref, dwk_ref, dwb_ref, fcw_ref, fcb_ref, o_ref):
    x = x_ref[...]
    zpad = jnp.zeros((129, 32), jnp.float32)
    xp = jnp.concatenate([zpad, x, zpad], axis=0)
    wcol = jax.lax.broadcasted_iota(jnp.int32, (NPIX, 1), 0) % WW
    acc = jnp.zeros((NPIX, 32), jnp.float32)
    k = 0
    for dy in (-1, 0, 1):
        for dx in (-1, 0, 1):
            s = dy * WW + dx
            sh = jax.lax.slice(xp, (129 + s, 0), (129 + s + NPIX, 32))
            if dx == -1:
                sh = jnp.where(wcol >= 1, sh, 0.0)
            elif dx == 1:
                sh = jnp.where(wcol <= WW - 2, sh, 0.0)
            acc = acc + sh * dwk_ref[k, :][None, :]
            k += 1
    y = acc + dwb_ref[...]
    y = jnp.where(y >= 0, y, 0.01 * y)
    logits = jnp.dot(y, fcw_ref[...], preferred_element_type=jnp.float32)
    logits = logits + fcb_ref[...]
    m = jnp.max(logits, axis=1, keepdims=True)
    e = jnp.exp(logits - m)
    o_ref[...] = e / jnp.sum(e, axis=1, keepdims=True)


def _head(X1, dwk, dwb, fcw, fcb):
    return pl.pallas_call(
        _head_kernel,
        out_shape=jax.ShapeDtypeStruct((NPIX, 16), jnp.float32),
    )(X1, dwk, dwb, fcw, fcb)


def kernel(A1, Q1, A2, Q2, S, W1, b1, W2, b2, bn_gamma, bn_beta, bn_mean,
           bn_var, pw_w, dw_w, dw_b, fc_w, fc_b):
    Wcat = jnp.concatenate([W1[0], W1[1], W1[2]], axis=1)
    bcat = jnp.reshape(b1, (1, 192))
    W2cat = jnp.concatenate([W2[0], W2[1], W2[2]], axis=1)

    def branch_sparse(A, Q):
        s0, H = _input_transform(Q, Wcat, bcat)
        Y1, s2 = _stage(A, H)
        return jnp.concatenate([s0, Y1[:, :64], s2], axis=1)

    f11 = branch_sparse(A1, Q1)
    f21 = branch_sparse(A2, Q2)
    amp = _amp(f11, f21)

    def branch_dense(A, f):
        d0, Gh = _fg(f, amp, W2cat)
        Y3, d2 = _stage(A, Gh)
        return jnp.concatenate([d0, Y3[:, :64], d2], axis=1)

    f12 = branch_dense(A1, f11)
    f22 = branch_dense(A2, f21)
    feats0 = jnp.concatenate([f12, f22], axis=1)

    scale = bn_gamma / jnp.sqrt(bn_var + 1e-5)
    shift = bn_beta - bn_mean * scale
    pwT = pw_w[:, :, 0, 0].T
    Wp = scale[:, None] * pwT
    bvec = jnp.concatenate([jnp.reshape(b2, (192,))] * 2)[None, :]
    cp_total = bvec @ Wp + shift[None, :] @ pwT

    F2 = _feats_project(feats0, Wp)
    X1 = _s_matmul(S, F2, cp_total)

    dwk = jnp.transpose(dw_w[:, 0], (1, 2, 0)).reshape(9, 32)
    return _head(X1, dwk, dw_b[None, :], fc_w, fc_b[None, :])


# speedup vs baseline: 1.0171x; 1.0171x over previous
import jax
import jax.numpy as jnp
from jax.experimental import pallas as pl
from jax.experimental.pallas import tpu as pltpu

N = 4096
F = 128
HH = 128
WW = 128
NPIX = HH * WW


def _input_kernel(x_ref, w_ref, b_ref, s0_ref, h_ref):
    acc = jnp.dot(x_ref[...], w_ref[...], preferred_element_type=jnp.float32)
    acc = jnp.maximum(acc + b_ref[...], 0.0)
    s0_ref[...] = acc[:, :64]
    h_ref[...] = acc[:, 64:].astype(jnp.bfloat16)


def _input_transform(Q, Wcat, bcat):
    BM = 512
    return pl.pallas_call(
        _input_kernel,
        grid=(N // BM,),
        in_specs=[
            pl.BlockSpec((BM, F), lambda i: (i, 0)),
            pl.BlockSpec((F, 192), lambda i: (0, 0)),
            pl.BlockSpec((1, 192), lambda i: (0, 0)),
        ],
        out_specs=[
            pl.BlockSpec((BM, 64), lambda i: (i, 0)),
            pl.BlockSpec((BM, 128), lambda i: (i, 0)),
        ],
        out_shape=[
            jax.ShapeDtypeStruct((N, 64), jnp.float32),
            jax.ShapeDtypeStruct((N, 128), jnp.bfloat16),
        ],
    )(Q, Wcat, bcat)


_BM = 256
_NB = N // _BM


def _stage_kernel(a_hbm, x_hbm, y_ref, z_ref, abf_ref, xbf_ref, tbf_ref,
                  abuf_ref, sems, semx):
    i = pl.program_id(0)

    @pl.when(i == 0)
    def _():
        cpx = pltpu.make_async_copy(x_hbm, xbf_ref, semx)
        cpx.start()
        pltpu.make_async_copy(a_hbm.at[pl.ds(0, _BM)], abuf_ref.at[0],
                              sems.at[0]).start()
        cpx.wait()

    @pl.when(i < _NB)
    def _():
        @pl.when(i + 1 < _NB)
        def _():
            nxt = (i + 1) % 2
            pltpu.make_async_copy(a_hbm.at[pl.ds((i + 1) * _BM, _BM)],
                                  abuf_ref.at[nxt], sems.at[nxt]).start()

        cur = i % 2
        pltpu.make_async_copy(a_hbm.at[pl.ds(i * _BM, _BM)],
                              abuf_ref.at[cur], sems.at[cur]).wait()
        ab = abuf_ref[cur].astype(jnp.bfloat16)
        abf_ref[pl.ds(i * _BM, _BM), :] = ab
        yb = jnp.dot(ab, xbf_ref[...], preferred_element_type=jnp.float32)
        y_ref[...] = yb
        tbf_ref[pl.ds(i * _BM, _BM), :] = yb[:, 64:].astype(jnp.bfloat16)

    @pl.when(i >= _NB)
    def _():
        j = i - _NB
        z_ref[...] = jnp.dot(abf_ref[pl.ds(j * _BM, _BM), :], tbf_ref[...],
                             preferred_element_type=jnp.float32)


def _stage(A, Xbf):
    return pl.pallas_call(
        _stage_kernel,
        grid=(2 * _NB,),
        in_specs=[
            pl.BlockSpec(memory_space=pl.ANY),
            pl.BlockSpec(memory_space=pl.ANY),
        ],
        out_specs=[
            pl.BlockSpec((_BM, 128), lambda i: (jnp.minimum(i, _NB - 1), 0)),
            pl.BlockSpec((_BM, 64), lambda i: (jnp.maximum(i - _NB, 0), 0)),
        ],
        out_shape=[
            jax.ShapeDtypeStruct((N, 128), jnp.float32),
            jax.ShapeDtypeStruct((N, 64), jnp.float32),
        ],
        scratch_shapes=[
            pltpu.VMEM((N, N), jnp.bfloat16),
            pltpu.VMEM((N, 128), jnp.bfloat16),
            pltpu.VMEM((N, 64), jnp.bfloat16),
            pltpu.VMEM((2, _BM, N), jnp.float32),
            pltpu.SemaphoreType.DMA((2,)),
            pltpu.SemaphoreType.DMA,
        ],
    )(A, Xbf)


def _amp_kernel(f1_ref, f2_ref, o_ref):
    f1 = f1_ref[...]
    f2 = f2_ref[...]
    n1 = jnp.maximum(jnp.sqrt(jnp.sum(f1 * f1, axis=0)), 1e-8)
    n2 = jnp.maximum(jnp.sqrt(jnp.sum(f2 * f2, axis=0)), 1e-8)
    cs = jnp.sum(f1 * f2, axis=0) / (n1 * n2)
    o_ref[...] = jax.nn.sigmoid(1.0 - cs)[None, :]


def _amp(f11, f21):
    return pl.pallas_call(
        _amp_kernel,
        out_shape=jax.ShapeDtypeStruct((1, 192), jnp.float32),
    )(f11, f21)


def _fg_kernel(f_ref, amp_ref, w_ref, g64_ref, gh_ref):
    acc = jnp.dot(f_ref[...] * amp_ref[...], w_ref[...],
                  preferred_element_type=jnp.float32)
    g64_ref[...] = acc[:, :64]
    gh_ref[...] = acc[:, 64:].astype(jnp.bfloat16)


def _fg(f, amp, W2cat):
    BM = 512
    return pl.pallas_call(
        _fg_kernel,
        grid=(N // BM,),
        in_specs=[
            pl.BlockSpec((BM, 192), lambda i: (i, 0)),
            pl.BlockSpec((1, 192), lambda i: (0, 0)),
            pl.BlockSpec((192, 192), lambda i: (0, 0)),
        ],
        out_specs=[
            pl.BlockSpec((BM, 64), lambda i: (i, 0)),
            pl.BlockSpec((BM, 128), lambda i: (i, 0)),
        ],
        out_shape=[
            jax.ShapeDtypeStruct((N, 64), jnp.float32),
            jax.ShapeDtypeStruct((N, 128), jnp.bfloat16),
        ],
    )(f, amp, W2cat)


def _mm_kernel(x_ref, w_ref, o_ref):
    o_ref[...] = jnp.dot(x_ref[...], w_ref[...], preferred_element_type=jnp.float32)


def _feats_project(feats0, Wp):
    BM = 512
    return pl.pallas_call(
        _mm_kernel,
        grid=(N // BM,),
        in_specs=[
            pl.BlockSpec((BM, 384), lambda i: (i, 0)),
            pl.BlockSpec((384, 32), lambda i: (0, 0)),
        ],
        out_specs=pl.BlockSpec((BM, 32), lambda i: (i, 0)),
        out_shape=jax.ShapeDtypeStruct((N, 32), jnp.float32),
    )(feats0, Wp)


def _s_kernel(s_ref, f_hbm, c_hbm, o_ref, fs_ref, cs_ref, sem1, sem2):
    i = pl.program_id(0)

    @pl.when(i == 0)
    def _():
        cp1 = pltpu.make_async_copy(f_hbm, fs_ref, sem1)
        cp1.start()
        cp2 = pltpu.make_async_copy(c_hbm, cs_ref, sem2)
        cp2.start()
        cp1.wait()
        cp2.wait()

    y = jnp.dot(s_ref[...], fs_ref[...], preferred_element_type=jnp.float32)
    y = y + cs_ref[...]
    o_ref[...] = jnp.where(y >= 0, y, 0.01 * y)


def _s_matmul(S, F2, cp):
    BM = 512
    return pl.pallas_call(
        _s_kernel,
        grid=(NPIX // BM,),
        in_specs=[
            pl.BlockSpec((BM, N), lambda i: (i, 0)),
            pl.BlockSpec(memory_space=pl.ANY),
            pl.BlockSpec(memory_space=pl.ANY),
        ],
        out_specs=pl.BlockSpec((BM, 32), lambda i: (i, 0)),
        out_shape=jax.ShapeDtypeStruct((NPIX, 32), jnp.float32),
        scratch_shapes=[
            pltpu.VMEM((N, 32), jnp.float32),
            pltpu.VMEM((1, 32), jnp.float32),
            pltpu.SemaphoreType.DMA,
            pltpu.SemaphoreType.DMA,
        ],
    )(S, F2, cp)


def _head_kernel(x_ref, dwk_ref, dwb_ref, fcw_ref, fcb_ref, o_ref):
    x = x_ref[...]
    zpad = jnp.zeros((129, 32), jnp.float32)
    xp = jnp.concatenate([zpad, x, zpad], axis=0)
    wcol = jax.lax.broadcasted_iota(jnp.int32, (NPIX, 1), 0) % WW
    acc = jnp.zeros((NPIX, 32), jnp.float32)
    k = 0
    for dy in (-1, 0, 1):
        for dx in (-1, 0, 1):
            s = dy * WW + dx
            sh = jax.lax.slice(xp, (129 + s, 0), (129 + s + NPIX, 32))
            if dx == -1:
                sh = jnp.where(wcol >= 1, sh, 0.0)
            elif dx == 1:
                sh = jnp.where(wcol <= WW - 2, sh, 0.0)
            acc = acc + sh * dwk_ref[k, :][None, :]
            k += 1
    y = acc + dwb_ref[...]
    y = jnp.where(y >= 0, y, 0.01 * y)
    logits = jnp.dot(y, fcw_ref[...], preferred_element_type=jnp.float32)
    logits = logits + fcb_ref[...]
    m = jnp.max(logits, axis=1, keepdims=True)
    e = jnp.exp(logits - m)
    o_ref[...] = e / jnp.sum(e, axis=1, keepdims=True)


def _head(X1, dwk, dwb, fcw, fcb):
    return pl.pallas_call(
        _head_kernel,
        out_shape=jax.ShapeDtypeStruct((NPIX, 16), jnp.float32),
    )(X1, dwk, dwb, fcw, fcb)


def kernel(A1, Q1, A2, Q2, S, W1, b1, W2, b2, bn_gamma, bn_beta, bn_mean,
           bn_var, pw_w, dw_w, dw_b, fc_w, fc_b):
    Wcat = jnp.concatenate([W1[0], W1[1], W1[2]], axis=1)
    bcat = jnp.reshape(b1, (1, 192))
    W2cat = jnp.concatenate([W2[0], W2[1], W2[2]], axis=1)

    def branch_sparse(A, Q):
        s0, H = _input_transform(Q, Wcat, bcat)
        Y1, s2 = _stage(A, H)
        return jnp.concatenate([s0, Y1[:, :64], s2], axis=1)

    f11 = branch_sparse(A1, Q1)
    f21 = branch_sparse(A2, Q2)
    amp = _amp(f11, f21)

    def branch_dense(A, f):
        d0, Gh = _fg(f, amp, W2cat)
        Y3, d2 = _stage(A, Gh)
        return jnp.concatenate([d0, Y3[:, :64], d2], axis=1)

    f12 = branch_dense(A1, f11)
    f22 = branch_dense(A2, f21)
    feats0 = jnp.concatenate([f12, f22], axis=1)

    scale = bn_gamma / jnp.sqrt(bn_var + 1e-5)
    shift = bn_beta - bn_mean * scale
    pwT = pw_w[:, :, 0, 0].T
    Wp = scale[:, None] * pwT
    bvec = jnp.concatenate([jnp.reshape(b2, (192,))] * 2)[None, :]
    cp_total = bvec @ Wp + shift[None, :] @ pwT

    F2 = _feats_project(feats0, Wp)
    X1 = _s_matmul(S, F2, cp_total)

    dwk = jnp.transpose(dw_w[:, 0], (1, 2, 0)).reshape(9, 32)
    return _head(X1, dwk, dw_b[None, :], fc_w, fc_b[None, :])


# speedup vs baseline: 1.0351x; 1.0177x over previous
import jax
import jax.numpy as jnp
from jax.experimental import pallas as pl
from jax.experimental.pallas import tpu as pltpu

N = 4096
F = 128
HH = 128
WW = 128
NPIX = HH * WW


def _input_kernel(x_ref, w_ref, b_ref, s0_ref, h_ref):
    acc = jnp.dot(x_ref[...], w_ref[...], preferred_element_type=jnp.float32)
    acc = jnp.maximum(acc + b_ref[...], 0.0)
    s0_ref[...] = acc[:, :64]
    h_ref[...] = acc[:, 64:].astype(jnp.bfloat16)


def _input_transform(Q, Wcat, bcat):
    BM = 512
    return pl.pallas_call(
        _input_kernel,
        grid=(N // BM,),
        in_specs=[
            pl.BlockSpec((BM, F), lambda i: (i, 0)),
            pl.BlockSpec((F, 192), lambda i: (0, 0)),
            pl.BlockSpec((1, 192), lambda i: (0, 0)),
        ],
        out_specs=[
            pl.BlockSpec((BM, 64), lambda i: (i, 0)),
            pl.BlockSpec((BM, 128), lambda i: (i, 0)),
        ],
        out_shape=[
            jax.ShapeDtypeStruct((N, 64), jnp.float32),
            jax.ShapeDtypeStruct((N, 128), jnp.bfloat16),
        ],
    )(Q, Wcat, bcat)


_BM = 256
_NB = N // _BM


def _stage_kernel(a_ref, x_hbm, y_ref, z_ref, abf_ref, xbf_ref, sem):
    i = pl.program_id(0)

    @pl.when(i == 0)
    def _():
        cp = pltpu.make_async_copy(x_hbm, xbf_ref, sem)
        cp.start()
        cp.wait()

    @pl.when(i < _NB)
    def _():
        ab = a_ref[...].astype(jnp.bfloat16)
        abf_ref[pl.ds(i * _BM, _BM), :] = ab
        y_ref[pl.ds(i * _BM, _BM), :] = jnp.dot(
            ab, xbf_ref[...], preferred_element_type=jnp.float32)

    @pl.when(i == _NB)
    def _():
        t = y_ref[:, 64:128].astype(jnp.bfloat16)
        z_ref[...] = jnp.dot(abf_ref[...], t, preferred_element_type=jnp.float32)


def _stage(A, Xbf):
    return pl.pallas_call(
        _stage_kernel,
        grid=(_NB + 1,),
        in_specs=[
            pl.BlockSpec((_BM, N), lambda i: (jnp.minimum(i, _NB - 1), 0)),
            pl.BlockSpec(memory_space=pl.ANY),
        ],
        out_specs=[
            pl.BlockSpec((N, 128), lambda i: (0, 0)),
            pl.BlockSpec((N, 64), lambda i: (0, 0)),
        ],
        out_shape=[
            jax.ShapeDtypeStruct((N, 128), jnp.float32),
            jax.ShapeDtypeStruct((N, 64), jnp.float32),
        ],
        scratch_shapes=[
            pltpu.VMEM((N, N), jnp.bfloat16),
            pltpu.VMEM((N, 128), jnp.bfloat16),
            pltpu.SemaphoreType.DMA,
        ],
    )(A, Xbf)


def _amp_kernel(f1_ref, f2_ref, o_ref):
    f1 = f1_ref[...]
    f2 = f2_ref[...]
    n1 = jnp.maximum(jnp.sqrt(jnp.sum(f1 * f1, axis=0)), 1e-8)
    n2 = jnp.maximum(jnp.sqrt(jnp.sum(f2 * f2, axis=0)), 1e-8)
    cs = jnp.sum(f1 * f2, axis=0) / (n1 * n2)
    o_ref[...] = jax.nn.sigmoid(1.0 - cs)[None, :]


def _amp(f11, f21):
    return pl.pallas_call(
        _amp_kernel,
        out_shape=jax.ShapeDtypeStruct((1, 192), jnp.float32),
    )(f11, f21)


def _fg_kernel(f_ref, amp_ref, w_ref, g64_ref, gh_ref):
    acc = jnp.dot(f_ref[...] * amp_ref[...], w_ref[...],
                  preferred_element_type=jnp.float32)
    g64_ref[...] = acc[:, :64]
    gh_ref[...] = acc[:, 64:].astype(jnp.bfloat16)


def _fg(f, amp, W2cat):
    BM = 512
    return pl.pallas_call(
        _fg_kernel,
        grid=(N // BM,),
        in_specs=[
            pl.BlockSpec((BM, 192), lambda i: (i, 0)),
            pl.BlockSpec((1, 192), lambda i: (0, 0)),
            pl.BlockSpec((192, 192), lambda i: (0, 0)),
        ],
        out_specs=[
            pl.BlockSpec((BM, 64), lambda i: (i, 0)),
            pl.BlockSpec((BM, 128), lambda i: (i, 0)),
        ],
        out_shape=[
            jax.ShapeDtypeStruct((N, 64), jnp.float32),
            jax.ShapeDtypeStruct((N, 128), jnp.bfloat16),
        ],
    )(f, amp, W2cat)


def _mm_kernel(x_ref, w_ref, o_ref):
    o_ref[...] = jnp.dot(x_ref[...], w_ref[...], preferred_element_type=jnp.float32)


def _feats_project(feats0, Wp):
    BM = 512
    return pl.pallas_call(
        _mm_kernel,
        grid=(N // BM,),
        in_specs=[
            pl.BlockSpec((BM, 384), lambda i: (i, 0)),
            pl.BlockSpec((384, 32), lambda i: (0, 0)),
        ],
        out_specs=pl.BlockSpec((BM, 32), lambda i: (i, 0)),
        out_shape=jax.ShapeDtypeStruct((N, 32), jnp.float32),
    )(feats0, Wp)


def _s_kernel(s_ref, f_hbm, c_hbm, o_ref, fs_ref, cs_ref, sem1, sem2):
    i = pl.program_id(0)

    @pl.when(i == 0)
    def _():
        cp1 = pltpu.make_async_copy(f_hbm, fs_ref, sem1)
        cp1.start()
        cp2 = pltpu.make_async_copy(c_hbm, cs_ref, sem2)
        cp2.start()
        cp1.wait()
        cp2.wait()

    y = jnp.dot(s_ref[...], fs_ref[...], preferred_element_type=jnp.float32)
    y = y + cs_ref[...]
    o_ref[...] = jnp.where(y >= 0, y, 0.01 * y)


def _s_matmul(S, F2, cp):
    BM = 1024
    return pl.pallas_call(
        _s_kernel,
        grid=(NPIX // BM,),
        in_specs=[
            pl.BlockSpec((BM, N), lambda i: (i, 0)),
            pl.BlockSpec(memory_space=pl.ANY),
            pl.BlockSpec(memory_space=pl.ANY),
        ],
        out_specs=pl.BlockSpec((BM, 32), lambda i: (i, 0)),
        out_shape=jax.ShapeDtypeStruct((NPIX, 32), jnp.float32),
        scratch_shapes=[
            pltpu.VMEM((N, 32), jnp.float32),
            pltpu.VMEM((1, 32), jnp.float32),
            pltpu.SemaphoreType.DMA,
            pltpu.SemaphoreType.DMA,
        ],
    )(S, F2, cp)


def _head_kernel(x_ref, dwk_ref, dwb_ref, fcw_ref, fcb_ref, o_ref):
    x = x_ref[...]
    zpad = jnp.zeros((129, 32), jnp.float32)
    xp = jnp.concatenate([zpad, x, zpad], axis=0)
    wcol = jax.lax.broadcasted_iota(jnp.int32, (NPIX, 1), 0) % WW
    acc = jnp.zeros((NPIX, 32), jnp.float32)
    k = 0
    for dy in (-1, 0, 1):
        for dx in (-1, 0, 1):
            s = dy * WW + dx
            sh = jax.lax.slice(xp, (129 + s, 0), (129 + s + NPIX, 32))
            if dx == -1:
                sh = jnp.where(wcol >= 1, sh, 0.0)
            elif dx == 1:
                sh = jnp.where(wcol <= WW - 2, sh, 0.0)
            acc = acc + sh * dwk_ref[k, :][None, :]
            k += 1
    y = acc + dwb_ref[...]
    y = jnp.where(y >= 0, y, 0.01 * y)
    logits = jnp.dot(y, fcw_ref[...], preferred_element_type=jnp.float32)
    logits = logits + fcb_ref[...]
    m = jnp.max(logits, axis=1, keepdims=True)
    e = jnp.exp(logits - m)
    o_ref[...] = e / jnp.sum(e, axis=1, keepdims=True)


def _head(X1, dwk, dwb, fcw, fcb):
    return pl.pallas_call(
        _head_kernel,
        out_shape=jax.ShapeDtypeStruct((NPIX, 16), jnp.float32),
    )(X1, dwk, dwb, fcw, fcb)


def kernel(A1, Q1, A2, Q2, S, W1, b1, W2, b2, bn_gamma, bn_beta, bn_mean,
           bn_var, pw_w, dw_w, dw_b, fc_w, fc_b):
    Wcat = jnp.concatenate([W1[0], W1[1], W1[2]], axis=1)
    bcat = jnp.reshape(b1, (1, 192))
    W2cat = jnp.concatenate([W2[0], W2[1], W2[2]], axis=1)

    def branch_sparse(A, Q):
        s0, H = _input_transform(Q, Wcat, bcat)
        Y1, s2 = _stage(A, H)
        return jnp.concatenate([s0, Y1[:, :64], s2], axis=1)

    f11 = branch_sparse(A1, Q1)
    f21 = branch_sparse(A2, Q2)
    amp = _amp(f11, f21)

    def branch_dense(A, f):
        d0, Gh = _fg(f, amp, W2cat)
        Y3, d2 = _stage(A, Gh)
        return jnp.concatenate([d0, Y3[:, :64], d2], axis=1)

    f12 = branch_dense(A1, f11)
    f22 = branch_dense(A2, f21)
    feats0 = jnp.concatenate([f12, f22], axis=1)

    scale = bn_gamma / jnp.sqrt(bn_var + 1e-5)
    shift = bn_beta - bn_mean * scale
    pwT = pw_w[:, :, 0, 0].T
    Wp = scale[:, None] * pwT
    bvec = jnp.concatenate([jnp.reshape(b2, (192,))] * 2)[None, :]
    cp_total = bvec @ Wp + shift[None, :] @ pwT

    F2 = _feats_project(feats0, Wp)
    X1 = _s_matmul(S, F2, cp_total)

    dwk = jnp.transpose(dw_w[:, 0], (1, 2, 0)).reshape(9, 32)
    return _head(X1, dwk, dw_b[None, :], fc_w, fc_b[None, :])


# speedup vs baseline: 1.1196x; 1.0816x over previous
import jax
import jax.numpy as jnp
from jax.experimental import pallas as pl
from jax.experimental.pallas import tpu as pltpu

N = 4096
F = 128
HH = 128
WW = 128
NPIX = HH * WW


def _input_kernel(x_ref, w_ref, b_ref, s0_ref, h_ref):
    acc = jnp.dot(x_ref[...], w_ref[...], preferred_element_type=jnp.float32)
    acc = jnp.maximum(acc + b_ref[...], 0.0)
    s0_ref[...] = acc[:, :64]
    h_ref[...] = acc[:, 64:].astype(jnp.bfloat16)


def _input_transform(Q, Wcat, bcat):
    BM = 512
    return pl.pallas_call(
        _input_kernel,
        grid=(N // BM,),
        in_specs=[
            pl.BlockSpec((BM, F), lambda i: (i, 0)),
            pl.BlockSpec((F, 192), lambda i: (0, 0)),
            pl.BlockSpec((1, 192), lambda i: (0, 0)),
        ],
        out_specs=[
            pl.BlockSpec((BM, 64), lambda i: (i, 0)),
            pl.BlockSpec((BM, 128), lambda i: (i, 0)),
        ],
        out_shape=[
            jax.ShapeDtypeStruct((N, 64), jnp.float32),
            jax.ShapeDtypeStruct((N, 128), jnp.bfloat16),
        ],
    )(Q, Wcat, bcat)


_BM = 256
_NB = N // _BM


def _stage_kernel(a_ref, x_hbm, y_ref, z_ref, abf_ref, xbf_ref, sem):
    i = pl.program_id(0)

    @pl.when(i == 0)
    def _():
        cp = pltpu.make_async_copy(x_hbm, xbf_ref, sem)
        cp.start()
        cp.wait()

    @pl.when(i < _NB)
    def _():
        ab = a_ref[...].astype(jnp.bfloat16)
        abf_ref[pl.ds(i * _BM, _BM), :] = ab
        y_ref[pl.ds(i * _BM, _BM), :] = jnp.dot(
            ab, xbf_ref[...], preferred_element_type=jnp.float32)

    @pl.when(i == _NB)
    def _():
        t = y_ref[:, 64:128].astype(jnp.bfloat16)
        z_ref[...] = jnp.dot(abf_ref[...], t, preferred_element_type=jnp.float32)


def _stage(A, Xbf):
    return pl.pallas_call(
        _stage_kernel,
        grid=(_NB + 1,),
        in_specs=[
            pl.BlockSpec((_BM, N), lambda i: (jnp.minimum(i, _NB - 1), 0)),
            pl.BlockSpec(memory_space=pl.ANY),
        ],
        out_specs=[
            pl.BlockSpec((N, 128), lambda i: (0, 0)),
            pl.BlockSpec((N, 64), lambda i: (0, 0)),
        ],
        out_shape=[
            jax.ShapeDtypeStruct((N, 128), jnp.float32),
            jax.ShapeDtypeStruct((N, 64), jnp.float32),
        ],
        scratch_shapes=[
            pltpu.VMEM((N, N), jnp.bfloat16),
            pltpu.VMEM((N, 128), jnp.bfloat16),
            pltpu.SemaphoreType.DMA,
        ],
    )(A, Xbf)


def _mid_kernel(s01_ref, y1_ref, s21_ref, s02_ref, y2_ref, s22_ref, w_ref,
                d01_ref, gh1_ref, d02_ref, gh2_ref):
    f11 = jnp.concatenate(
        [s01_ref[...], y1_ref[:, :64], s21_ref[...]], axis=1)
    f21 = jnp.concatenate(
        [s02_ref[...], y2_ref[:, :64], s22_ref[...]], axis=1)
    n1 = jnp.maximum(jnp.sqrt(jnp.sum(f11 * f11, axis=0)), 1e-8)
    n2 = jnp.maximum(jnp.sqrt(jnp.sum(f21 * f21, axis=0)), 1e-8)
    cs = jnp.sum(f11 * f21, axis=0) / (n1 * n2)
    amp = jax.nn.sigmoid(1.0 - cs)[None, :]
    g1 = jnp.dot(f11 * amp, w_ref[...], preferred_element_type=jnp.float32)
    g2 = jnp.dot(f21 * amp, w_ref[...], preferred_element_type=jnp.float32)
    d01_ref[...] = g1[:, :64]
    gh1_ref[...] = g1[:, 64:].astype(jnp.bfloat16)
    d02_ref[...] = g2[:, :64]
    gh2_ref[...] = g2[:, 64:].astype(jnp.bfloat16)


def _mid(s01, Y1, s21, s02, Y2, s22, W2cat):
    return pl.pallas_call(
        _mid_kernel,
        out_shape=[
            jax.ShapeDtypeStruct((N, 64), jnp.float32),
            jax.ShapeDtypeStruct((N, 128), jnp.bfloat16),
            jax.ShapeDtypeStruct((N, 64), jnp.float32),
            jax.ShapeDtypeStruct((N, 128), jnp.bfloat16),
        ],
    )(s01, Y1, s21, s02, Y2, s22, W2cat)


def _fp_kernel(a_ref, b_ref, c_ref, d_ref, e_ref, f_ref, w_ref, o_ref):
    feats = jnp.concatenate(
        [a_ref[...], b_ref[:, :64], c_ref[...],
         d_ref[...], e_ref[:, :64], f_ref[...]], axis=1)
    o_ref[...] = jnp.dot(feats, w_ref[...], preferred_element_type=jnp.float32)


def _feats_project(d01, Y31, d21, d02, Y32, d22, Wp):
    BM = 512
    blk64 = pl.BlockSpec((BM, 64), lambda i: (i, 0))
    blk128 = pl.BlockSpec((BM, 128), lambda i: (i, 0))
    return pl.pallas_call(
        _fp_kernel,
        grid=(N // BM,),
        in_specs=[
            blk64, blk128, blk64, blk64, blk128, blk64,
            pl.BlockSpec((384, 32), lambda i: (0, 0)),
        ],
        out_specs=pl.BlockSpec((BM, 32), lambda i: (i, 0)),
        out_shape=jax.ShapeDtypeStruct((N, 32), jnp.float32),
    )(d01, Y31, d21, d02, Y32, d22, Wp)


def _s_kernel(s_ref, f_hbm, c_hbm, o_ref, fs_ref, cs_ref, sem1, sem2):
    i = pl.program_id(0)

    @pl.when(i == 0)
    def _():
        cp1 = pltpu.make_async_copy(f_hbm, fs_ref, sem1)
        cp1.start()
        cp2 = pltpu.make_async_copy(c_hbm, cs_ref, sem2)
        cp2.start()
        cp1.wait()
        cp2.wait()

    y = jnp.dot(s_ref[...], fs_ref[...], preferred_element_type=jnp.float32)
    y = y + cs_ref[...]
    o_ref[...] = jnp.where(y >= 0, y, 0.01 * y)


def _s_matmul(S, F2, cp):
    BM = 1024
    return pl.pallas_call(
        _s_kernel,
        grid=(NPIX // BM,),
        in_specs=[
            pl.BlockSpec((BM, N), lambda i: (i, 0)),
            pl.BlockSpec(memory_space=pl.ANY),
            pl.BlockSpec(memory_space=pl.ANY),
        ],
        out_specs=pl.BlockSpec((BM, 32), lambda i: (i, 0)),
        out_shape=jax.ShapeDtypeStruct((NPIX, 32), jnp.float32),
        scratch_shapes=[
            pltpu.VMEM((N, 32), jnp.float32),
            pltpu.VMEM((1, 32), jnp.float32),
            pltpu.SemaphoreType.DMA,
            pltpu.SemaphoreType.DMA,
        ],
    )(S, F2, cp)


def _head_kernel(x_ref, dwk_ref, dwb_ref, fcw_ref, fcb_ref, o_ref):
    x = x_ref[...]
    zpad = jnp.zeros((129, 32), jnp.float32)
    xp = jnp.concatenate([zpad, x, zpad], axis=0)
    wcol = jax.lax.broadcasted_iota(jnp.int32, (NPIX, 1), 0) % WW
    acc = jnp.zeros((NPIX, 32), jnp.float32)
    k = 0
    for dy in (-1, 0, 1):
        for dx in (-1, 0, 1):
            s = dy * WW + dx
            sh = jax.lax.slice(xp, (129 + s, 0), (129 + s + NPIX, 32))
            if dx == -1:
                sh = jnp.where(wcol >= 1, sh, 0.0)
            elif dx == 1:
                sh = jnp.where(wcol <= WW - 2, sh, 0.0)
            acc = acc + sh * dwk_ref[k, :][None, :]
            k += 1
    y = acc + dwb_ref[...]
    y = jnp.where(y >= 0, y, 0.01 * y)
    logits = jnp.dot(y, fcw_ref[...], preferred_element_type=jnp.float32)
    logits = logits + fcb_ref[...]
    m = jnp.max(logits, axis=1, keepdims=True)
    e = jnp.exp(logits - m)
    o_ref[...] = e / jnp.sum(e, axis=1, keepdims=True)


def _head(X1, dwk, dwb, fcw, fcb):
    return pl.pallas_call(
        _head_kernel,
        out_shape=jax.ShapeDtypeStruct((NPIX, 16), jnp.float32),
    )(X1, dwk, dwb, fcw, fcb)


def kernel(A1, Q1, A2, Q2, S, W1, b1, W2, b2, bn_gamma, bn_beta, bn_mean,
           bn_var, pw_w, dw_w, dw_b, fc_w, fc_b):
    Wcat = jnp.concatenate([W1[0], W1[1], W1[2]], axis=1)
    bcat = jnp.reshape(b1, (1, 192))
    W2cat = jnp.concatenate([W2[0], W2[1], W2[2]], axis=1)

    s01, H1 = _input_transform(Q1, Wcat, bcat)
    s02, H2 = _input_transform(Q2, Wcat, bcat)
    Y1, s21 = _stage(A1, H1)
    Y2, s22 = _stage(A2, H2)
    d01, Gh1, d02, Gh2 = _mid(s01, Y1, s21, s02, Y2, s22, W2cat)
    Y31, d21 = _stage(A1, Gh1)
    Y32, d22 = _stage(A2, Gh2)

    scale = bn_gamma / jnp.sqrt(bn_var + 1e-5)
    shift = bn_beta - bn_mean * scale
    pwT = pw_w[:, :, 0, 0].T
    Wp = scale[:, None] * pwT
    bvec = jnp.concatenate([jnp.reshape(b2, (192,))] * 2)[None, :]
    cp_total = bvec @ Wp + shift[None, :] @ pwT

    F2 = _feats_project(d01, Y31, d21, d02, Y32, d22, Wp)
    X1 = _s_matmul(S, F2, cp_total)

    dwk = jnp.transpose(dw_w[:, 0], (1, 2, 0)).reshape(9, 32)
    return _head(X1, dwk, dw_b[None, :], fc_w, fc_b[None, :])
